# baseline (speedup 1.0000x reference)
import functools

import numpy as np
import jax
import jax.numpy as jnp
from jax import lax
from jax.experimental import pallas as pl
from jax.experimental.pallas import tpu as pltpu
from jax.experimental.pallas import tpu_sc as plsc

_FIELD_DIMS = [1000] * 26
_F = 26
_FEAT = sum(_FIELD_DIMS)
_D = 16
_PAIRS = _F * (_F - 1) // 2
_INTER = _PAIRS * _D
_INTER_PAD = 5248
_B = 4096
_OFFS = np.asarray([0, *np.cumsum(_FIELD_DIMS)[:-1]], dtype=np.int32)

_NIDX = _F * _F
_NIDX_PAD = 680
_CHUNKS = (128, 128, 128, 128, 128, 40)

_NW = 32
_BPW = _B // _NW


def _sc_make():
    mesh = plsc.VectorSubcoreMesh(core_axis_name="c", subcore_axis_name="s")

    @functools.partial(
        pl.kernel,
        mesh=mesh,
        compiler_params=pltpu.CompilerParams(use_tc_tiling_on_sc=False),
        out_type=[
            jax.ShapeDtypeStruct((_B, _INTER_PAD), jnp.float32),
            jax.ShapeDtypeStruct((_B, 16), jnp.float32),
        ],
        scratch_types=[
            pltpu.VMEM((2, _NIDX_PAD), jnp.int32),
            pltpu.VMEM((_F, _BPW), jnp.int32),
            pltpu.VMEM((2, _NIDX_PAD, _D), jnp.float32),
            pltpu.VMEM((2, _INTER_PAD), jnp.float32),
            pltpu.VMEM((_F, _BPW, _D), jnp.float32),
            pltpu.VMEM((_BPW, 16), jnp.float32),
            pltpu.SemaphoreType.DMA,
            pltpu.SemaphoreType.DMA,
            pltpu.SemaphoreType.DMA,
            pltpu.SemaphoreType.DMA,
            pltpu.SemaphoreType.DMA,
            pltpu.SemaphoreType.DMA,
        ],
    )
    def sc_interactions(emb_hbm, xfull_hbm, xoffT_hbm, wpad_hbm,
                        h_hbm, fo_hbm,
                        ix_v, idx_v, R_v, h_v, w_all, fo_v,
                        isem0, isem1, gsem0, gsem1, wsem0, wsem1):
        wid = lax.axis_index("s") * 2 + lax.axis_index("c")
        base = wid * _BPW

        pltpu.sync_copy(xoffT_hbm.at[:, pl.ds(base, _BPW)], idx_v)
        for f in range(_F):
            pltpu.async_copy(wpad_hbm.at[idx_v.at[f]], w_all.at[f], wsem0)
        for f in range(_F):
            pltpu.make_async_copy(
                wpad_hbm.at[idx_v.at[f]], w_all.at[f], wsem0).wait()

        zero16 = jnp.zeros((16,), jnp.float32)
        for slot in (0, 1):
            for j in range(3):
                h_v[slot, pl.ds(_INTER + j * 16, 16)] = zero16

        def fire_idx(b, slot, isem):
            pltpu.async_copy(xfull_hbm.at[b + base], ix_v.at[slot], isem)

        def wait_idx(b, slot, isem):
            pltpu.make_async_copy(
                xfull_hbm.at[b + base], ix_v.at[slot], isem).wait()

        def fire_gather(slot, gsem):
            off = 0
            for sz in _CHUNKS:
                pltpu.async_copy(
                    emb_hbm.at[ix_v.at[slot, pl.ds(off, sz)]],
                    R_v.at[slot, pl.ds(off, sz)], gsem)
                off += sz

        def wait_gather(slot, gsem):
            off = 0
            for sz in _CHUNKS:
                pltpu.make_async_copy(
                    emb_hbm.at[ix_v.at[slot, pl.ds(off, sz)]],
                    R_v.at[slot, pl.ds(off, sz)], gsem).wait()
                off += sz

        fire_idx(0, 0, isem0)
        fire_idx(1, 1, isem1)
        wait_idx(0, 0, isem0)
        fire_gather(0, gsem0)

        def half_step(b, slot, o_slot, gsem, o_isem, o_gsem, wsem):
            wait_gather(slot, gsem)
            @pl.when(b + 2 < _BPW)
            def _():
                fire_idx(b + 2, slot, [isem0, isem1][slot])
            @pl.when(b + 1 < _BPW)
            def _():
                wait_idx(b + 1, o_slot, o_isem)
                fire_gather(o_slot, o_gsem)
            @pl.when(b >= 2)
            def _():
                pltpu.make_async_copy(
                    h_v.at[slot], h_hbm.at[base + b - 2], wsem).wait()

            def f_loop(f, p):
                aj0 = f * _F
                def g_loop(g, p):
                    va = R_v[slot, aj0 + g, :]
                    vb = R_v[slot, g * _F + f, :]
                    h_v[slot, pl.ds(pl.multiple_of(p, 16), 16)] = va * vb
                    return p + 16
                return lax.fori_loop(f + 1, _F, g_loop, p)
            lax.fori_loop(0, _F - 1, f_loop, 0)

            def w_loop(f, acc):
                return acc + w_all[f, b, :]
            fo_v[b, :] = lax.fori_loop(
                0, _F, w_loop, jnp.zeros((16,), jnp.float32))

            pltpu.async_copy(h_v.at[slot], h_hbm.at[base + b], wsem)

        def iter_body(i, _):
            b0 = 2 * i
            half_step(b0, 0, 1, gsem0, isem1, gsem1, wsem0)
            half_step(b0 + 1, 1, 0, gsem1, isem0, gsem0, wsem1)
            return 0
        lax.fori_loop(0, _BPW // 2, iter_body, 0)

        pltpu.make_async_copy(
            h_v.at[0], h_hbm.at[base + _BPW - 2], wsem0).wait()
        pltpu.make_async_copy(
            h_v.at[1], h_hbm.at[base + _BPW - 1], wsem1).wait()

        pltpu.sync_copy(fo_v, fo_hbm.at[pl.ds(base, _BPW)])

    return sc_interactions


_sc_interactions = _sc_make()

_BT = 512


def _mlp_body(h_ref, fo_ref, W1_ref, b1_ref, W2_ref, b2_ref, W3_ref, b3_ref,
              out_ref):
    a1 = jnp.dot(h_ref[...], W1_ref[...], preferred_element_type=jnp.float32)
    a1 = jnp.maximum(a1 + b1_ref[...], 0.0)
    a2 = jnp.dot(a1, W2_ref[...], preferred_element_type=jnp.float32)
    a2 = jnp.maximum(a2 + b2_ref[...], 0.0)
    a3 = jnp.dot(a2, W3_ref[...], preferred_element_type=jnp.float32)
    fo = jnp.sum(fo_ref[...], axis=1, keepdims=True)
    out_ref[...] = a3 + fo + b3_ref[...]


_mlp_call = pl.pallas_call(
    _mlp_body,
    grid=(_B // _BT,),
    in_specs=[
        pl.BlockSpec((_BT, _INTER_PAD), lambda i: (i, 0)),
        pl.BlockSpec((_BT, 16), lambda i: (i, 0)),
        pl.BlockSpec((_INTER_PAD, 64), lambda i: (0, 0)),
        pl.BlockSpec((1, 64), lambda i: (0, 0)),
        pl.BlockSpec((64, 32), lambda i: (0, 0)),
        pl.BlockSpec((1, 32), lambda i: (0, 0)),
        pl.BlockSpec((32, 1), lambda i: (0, 0)),
        pl.BlockSpec((1, 1), lambda i: (0, 0)),
    ],
    out_specs=pl.BlockSpec((_BT, 1), lambda i: (i, 0)),
    out_shape=jax.ShapeDtypeStruct((_B, 1), jnp.float32),
)


def kernel(x, emb, w_lin, b_lin, W1, b1, W2, b2, W3, b3):
    x_off = x + jnp.asarray(_OFFS)[None, :]
    emb_flat = emb.reshape(_F * _FEAT, _D)
    fbase = (jnp.arange(_F, dtype=jnp.int32) * _FEAT)[None, :, None]
    x_full = (x_off[:, None, :] + fbase).reshape(_B, _NIDX)
    x_full = jnp.concatenate(
        [x_full, jnp.zeros((_B, _NIDX_PAD - _NIDX), jnp.int32)], axis=1)
    w_pad = jnp.concatenate(
        [w_lin.reshape(_FEAT, 1), jnp.zeros((_FEAT, 15), jnp.float32)], axis=1)
    h, fo = _sc_interactions(emb_flat, x_full, x_off.T, w_pad)
    W1p = jnp.concatenate(
        [W1, jnp.zeros((_INTER_PAD - _INTER, 64), jnp.float32)], axis=0)
    out = _mlp_call(h, fo, W1p, b1.reshape(1, 64), W2, b2.reshape(1, 32),
                    W3, (b3 + b_lin).reshape(1, 1))
    return out[:, 0]
